# baseline (speedup 1.0000x reference)
import jax
import jax.numpy as jnp
from jax import lax
from jax.experimental import pallas as pl
from jax.experimental.pallas import tpu as pltpu

NUM_POS = 47360
NUM_NEG = 236800
EMB = 128
GRID = 40
BP = NUM_POS // GRID
BN = NUM_NEG // GRID

_DN = (((1,), (1,)), ((), ()))


def _body(pu, pv, nu, nv, out_ref, accp_ref, accn_ref):
    i = pl.program_id(0)

    @pl.when(i == 0)
    def _():
        accp_ref[...] = jnp.zeros_like(accp_ref)
        accn_ref[...] = jnp.zeros_like(accn_ref)

    ones = jnp.ones((1, EMB), jnp.float32)

    p = pu[...] * pv[...]
    sp = lax.dot_general(ones, p, _DN, preferred_element_type=jnp.float32)
    sp = jnp.clip(sp, -6.0, 6.0)
    accp_ref[...] += jnp.log1p(jnp.exp(-sp))

    n = nu[...] * nv[...]
    sn = lax.dot_general(ones, n, _DN, preferred_element_type=jnp.float32)
    sn = jnp.clip(sn, -6.0, 6.0)
    accn_ref[...] += jnp.log1p(jnp.exp(sn))

    @pl.when(i == GRID - 1)
    def _():
        out_ref[0] = (jnp.sum(accp_ref[...]) * (1.0 / NUM_POS)
                      + jnp.sum(accn_ref[...]) * (1.0 / NUM_NEG))


def kernel(emb_pos_u, emb_pos_v, emb_neg_u, emb_neg_v):
    loss = pl.pallas_call(
        _body,
        grid=(GRID,),
        in_specs=[
            pl.BlockSpec((BP, EMB), lambda i: (i, 0)),
            pl.BlockSpec((BP, EMB), lambda i: (i, 0)),
            pl.BlockSpec((BN, EMB), lambda i: (i, 0)),
            pl.BlockSpec((BN, EMB), lambda i: (i, 0)),
        ],
        out_specs=pl.BlockSpec(memory_space=pltpu.MemorySpace.SMEM),
        out_shape=jax.ShapeDtypeStruct((1,), jnp.float32),
        scratch_shapes=[
            pltpu.VMEM((1, BP), jnp.float32),
            pltpu.VMEM((1, BN), jnp.float32),
        ],
    )(emb_pos_u, emb_pos_v, emb_neg_u, emb_neg_v)
    return loss[0]


# speedup vs baseline: 1.0029x; 1.0029x over previous
import jax
import jax.numpy as jnp
from jax import lax
from jax.experimental import pallas as pl
from jax.experimental.pallas import tpu as pltpu

NUM_POS = 47360
NUM_NEG = 236800
EMB = 128
GRID = 37
BP = NUM_POS // GRID
BN = NUM_NEG // GRID

_DN = (((1,), (1,)), ((), ()))


def _body(pu, pv, nu, nv, out_ref, accp_ref, accn_ref):
    i = pl.program_id(0)

    @pl.when(i == 0)
    def _():
        accp_ref[...] = jnp.zeros_like(accp_ref)
        accn_ref[...] = jnp.zeros_like(accn_ref)

    ones = jnp.ones((1, EMB), jnp.float32)

    p = pu[...] * pv[...]
    sp = lax.dot_general(ones, p, _DN, preferred_element_type=jnp.float32)
    sp = jnp.clip(sp, -6.0, 6.0)
    accp_ref[...] += jnp.log1p(jnp.exp(-sp))

    n = nu[...] * nv[...]
    sn = lax.dot_general(ones, n, _DN, preferred_element_type=jnp.float32)
    sn = jnp.clip(sn, -6.0, 6.0)
    accn_ref[...] += jnp.log1p(jnp.exp(sn))

    @pl.when(i == GRID - 1)
    def _():
        out_ref[0] = (jnp.sum(accp_ref[...]) * (1.0 / NUM_POS)
                      + jnp.sum(accn_ref[...]) * (1.0 / NUM_NEG))


def kernel(emb_pos_u, emb_pos_v, emb_neg_u, emb_neg_v):
    loss = pl.pallas_call(
        _body,
        grid=(GRID,),
        in_specs=[
            pl.BlockSpec((BP, EMB), lambda i: (i, 0)),
            pl.BlockSpec((BP, EMB), lambda i: (i, 0)),
            pl.BlockSpec((BN, EMB), lambda i: (i, 0)),
            pl.BlockSpec((BN, EMB), lambda i: (i, 0)),
        ],
        out_specs=pl.BlockSpec(memory_space=pltpu.MemorySpace.SMEM),
        out_shape=jax.ShapeDtypeStruct((1,), jnp.float32),
        scratch_shapes=[
            pltpu.VMEM((1, BP), jnp.float32),
            pltpu.VMEM((1, BN), jnp.float32),
        ],
    )(emb_pos_u, emb_pos_v, emb_neg_u, emb_neg_v)
    return loss[0]
